# Initial kernel scaffold; baseline (speedup 1.0000x reference)
import jax
import jax.numpy as jnp
from jax.experimental import pallas as pl


def kernel(hidden_states, router_weight, gate_up_proj, down_proj, shared_gate_w, shared_up_w, shared_down_w):
    raise NotImplementedError("write your pallas kernel here")


# speedup vs baseline: 1.0136x; 1.0136x over previous
import functools

import jax
import jax.numpy as jnp
from jax import lax
from jax.experimental import pallas as pl
from jax.experimental.pallas import tpu as pltpu
from jax.experimental.pallas import tpu_sc as plsc

H = 2048
I = 2048
NE = 8
T = 2048
TM = 128
P = T + NE * TM
NP = P // TM
TN = 1024
NW = 32
CH = 16


def _router_body(x_ref, w_ref, logits_ref, eid_ref, score_ref):
    logits = jnp.dot(x_ref[...], w_ref[...], preferred_element_type=jnp.float32)
    logits_ref[...] = logits
    eid = jnp.argmax(logits, axis=1).astype(jnp.int32)
    eid_ref[...] = eid[:, None]
    score_ref[...] = jax.nn.sigmoid(jnp.max(logits, axis=1))[:, None]


def _router(x, rw):
    return pl.pallas_call(
        _router_body,
        grid=(T // TM,),
        in_specs=[
            pl.BlockSpec((TM, H), lambda i: (i, 0)),
            pl.BlockSpec((H, NE), lambda i: (0, 0)),
        ],
        out_specs=[
            pl.BlockSpec((TM, NE), lambda i: (i, 0)),
            pl.BlockSpec((TM, 1), lambda i: (i, 0)),
            pl.BlockSpec((TM, 1), lambda i: (i, 0)),
        ],
        out_shape=[
            jax.ShapeDtypeStruct((T, NE), jnp.float32),
            jax.ShapeDtypeStruct((T, 1), jnp.int32),
            jax.ShapeDtypeStruct((T, 1), jnp.float32),
        ],
    )(x, rw)


def _metadata(eid, score):
    onehot = (eid[:, None] == jnp.arange(NE, dtype=jnp.int32)[None, :]).astype(jnp.int32)
    counts = onehot.sum(axis=0)
    tiles_e = (counts + TM - 1) // TM
    tile_bound = jnp.cumsum(tiles_e)
    row_start = (tile_bound - tiles_e) * TM
    rank = jnp.cumsum(onehot, axis=0) - onehot
    rank_t = jnp.take_along_axis(rank, eid[:, None], axis=1)[:, 0]
    pos = row_start[eid] + rank_t
    gather_idx = jnp.zeros((P,), jnp.int32).at[pos].set(
        jnp.arange(T, dtype=jnp.int32))
    score_sorted = jnp.zeros((P,), jnp.float32).at[pos].set(score)
    tile_gid = jnp.clip(
        jnp.searchsorted(tile_bound, jnp.arange(NP, dtype=jnp.int32),
                         side="right").astype(jnp.int32), 0, NE - 1)
    return pos, gather_idx, score_sorted, tile_gid


def _sc_gather_rows(table, idx):
    B = idx.shape[0]
    D = table.shape[1]
    b_per_w = B // NW
    n_chunks = b_per_w // CH
    mesh = plsc.VectorSubcoreMesh(core_axis_name="c", subcore_axis_name="s")

    @functools.partial(
        pl.kernel,
        out_type=jax.ShapeDtypeStruct((B, D), jnp.float32),
        mesh=mesh,
        scratch_types=[
            pltpu.VMEM((CH,), jnp.int32),
            pltpu.VMEM((CH, D), jnp.float32),
            pltpu.SemaphoreType.DMA,
        ],
    )
    def k(table_hbm, idx_hbm, out_hbm, idx_v, rows_v, sem):
        wid = lax.axis_index("s") * 2 + lax.axis_index("c")
        base = wid * b_per_w

        for c in range(n_chunks):
            off = base + c * CH
            pltpu.sync_copy(idx_hbm.at[pl.ds(off, CH)], idx_v)
            pltpu.async_copy(table_hbm.at[idx_v], rows_v, sem).wait()
            pltpu.sync_copy(rows_v, out_hbm.at[pl.ds(off, CH)])

    return k(table, idx)


def _gmm1_body(gid_ref, x_ref, s_ref, wg_ref, wu_ref, out_ref):
    xs = x_ref[...] * s_ref[...]
    g = jnp.dot(xs, wg_ref[0], preferred_element_type=jnp.float32)
    u = jnp.dot(xs, wu_ref[0], preferred_element_type=jnp.float32)
    out_ref[...] = g * jax.nn.sigmoid(g) * u


def _gmm1(x_sorted, score_sorted, wg, wu, tile_gid):
    grid = (I // TN, NP)
    return pl.pallas_call(
        _gmm1_body,
        grid_spec=pltpu.PrefetchScalarGridSpec(
            num_scalar_prefetch=1,
            grid=grid,
            in_specs=[
                pl.BlockSpec((TM, H), lambda n, i, gid: (i, 0)),
                pl.BlockSpec((TM, 1), lambda n, i, gid: (i, 0)),
                pl.BlockSpec((1, H, TN), lambda n, i, gid: (gid[i], 0, n)),
                pl.BlockSpec((1, H, TN), lambda n, i, gid: (gid[i], 0, n)),
            ],
            out_specs=pl.BlockSpec((TM, TN), lambda n, i, gid: (i, n)),
        ),
        out_shape=jax.ShapeDtypeStruct((P, I), jnp.float32),
    )(tile_gid, x_sorted, score_sorted[:, None], wg, wu)


def _gmm2_body(gid_ref, a_ref, wd_ref, out_ref):
    out_ref[...] = jnp.dot(a_ref[...], wd_ref[0],
                           preferred_element_type=jnp.float32)


def _gmm2(act_sorted, wd, tile_gid):
    grid = (H // TN, NP)
    return pl.pallas_call(
        _gmm2_body,
        grid_spec=pltpu.PrefetchScalarGridSpec(
            num_scalar_prefetch=1,
            grid=grid,
            in_specs=[
                pl.BlockSpec((TM, I), lambda n, i, gid: (i, 0)),
                pl.BlockSpec((1, I, TN), lambda n, i, gid: (gid[i], 0, n)),
            ],
            out_specs=pl.BlockSpec((TM, TN), lambda n, i, gid: (i, n)),
        ),
        out_shape=jax.ShapeDtypeStruct((P, H), jnp.float32),
    )(tile_gid, act_sorted, wd)


def _shared1_body(x_ref, gw_ref, uw_ref, out_ref):
    x = x_ref[...]
    g = jnp.dot(x, gw_ref[...], preferred_element_type=jnp.float32)
    u = jnp.dot(x, uw_ref[...], preferred_element_type=jnp.float32)
    out_ref[...] = g * jax.nn.sigmoid(g) * u


def _shared1(x, gw, uw):
    grid = (I // TN, T // TM)
    return pl.pallas_call(
        _shared1_body,
        grid=grid,
        in_specs=[
            pl.BlockSpec((TM, H), lambda n, i: (i, 0)),
            pl.BlockSpec((H, TN), lambda n, i: (0, n)),
            pl.BlockSpec((H, TN), lambda n, i: (0, n)),
        ],
        out_specs=pl.BlockSpec((TM, TN), lambda n, i: (i, n)),
        out_shape=jax.ShapeDtypeStruct((T, I), jnp.float32),
    )(x, gw, uw)


def _shared2_body(a_ref, dw_ref, y_ref, out_ref):
    out_ref[...] = y_ref[...] + jnp.dot(a_ref[...], dw_ref[...],
                                        preferred_element_type=jnp.float32)


def _shared2(act, dw, y_routed):
    grid = (H // TN, T // TM)
    return pl.pallas_call(
        _shared2_body,
        grid=grid,
        in_specs=[
            pl.BlockSpec((TM, I), lambda n, i: (i, 0)),
            pl.BlockSpec((I, TN), lambda n, i: (0, n)),
            pl.BlockSpec((TM, TN), lambda n, i: (i, n)),
        ],
        out_specs=pl.BlockSpec((TM, TN), lambda n, i: (i, n)),
        out_shape=jax.ShapeDtypeStruct((T, H), jnp.float32),
    )(act, dw, y_routed)


@jax.jit
def kernel(hidden_states, router_weight, gate_up_proj, down_proj,
           shared_gate_w, shared_up_w, shared_down_w):
    x = hidden_states.reshape(-1, H)

    logits, eid2d, score2d = _router(x, router_weight)
    pos, gather_idx, score_sorted, tile_gid = _metadata(
        eid2d[:, 0], score2d[:, 0])

    x_sorted = _sc_gather_rows(x, gather_idx)

    wg = gate_up_proj[:, :, :I]
    wu = gate_up_proj[:, :, I:]
    act_sorted = _gmm1(x_sorted, score_sorted, wg, wu, tile_gid)
    y_sorted = _gmm2(act_sorted, down_proj, tile_gid)

    y_routed = _sc_gather_rows(y_sorted, pos)

    act_s = _shared1(x, shared_gate_w, shared_up_w)
    out = _shared2(act_s, shared_down_w, y_routed)
    return out, logits
